# Initial kernel scaffold; baseline (speedup 1.0000x reference)
import jax
import jax.numpy as jnp
from jax.experimental import pallas as pl

_B, _N, _HID = 2, 2048, 768
_IDX_D = 64
_K = 10
_R = 4
_MEM_D = 64
_INTER = 3072
_DEPTH = 4
_P = 100
_PP = 128
_RATIO = 10

_BT = 256
_T = _B * _N
_GRID = _T // _BT

_HI = jax.lax.Precision.HIGHEST


def _dot(a, b, precision=_HI):
    return jax.lax.dot_general(a, b, (((1,), (0,)), ((), ())),
                               precision=precision,
                               preferred_element_type=jnp.float32)


def _dot_nt(a, b, precision=_HI):
    return jax.lax.dot_general(a, b, (((1,), (1,)), ((), ())),
                               precision=precision,
                               preferred_element_type=jnp.float32)


def _extract_topk(cand, codes, k):
    L = cand.shape[1]
    jiota = jax.lax.broadcasted_iota(jnp.int32, cand.shape, 1)
    vals, inds = [], []
    for _ in range(k):
        vmax = jnp.max(cand, axis=1, keepdims=True)
        is_max = cand == vmax
        jstar = jnp.min(jnp.where(is_max, jiota, L), axis=1, keepdims=True)
        oh = jiota == jstar
        code = jnp.sum(jnp.where(oh, codes, 0), axis=1, keepdims=True)
        vals.append(vmax)
        inds.append(code)
        cand = jnp.where(oh, -1.0, cand)
    return vals, inds


def _fused_kernel(x_ref, wq_ref, bq_ref, k0_ref, kr1_ref, kr2_ref, kr3_ref,
                  v_ref, w1_ref, b1_ref, w2_ref, b2_ref, g_ref, be_ref,
                  out_ref):
    x = x_ref[...]
    xq = _dot(x, wq_ref[...]) + bq_ref[...]

    piota = jax.lax.broadcasted_iota(jnp.int32, (_BT, _PP), 1)
    in_range = piota < _P

    q0 = xq[:, 0:_IDX_D]
    s0 = _dot_nt(q0, k0_ref[...]) * 0.125
    s0 = jnp.where(in_range, s0, -1e30)
    m0 = jnp.max(s0, axis=1, keepdims=True)
    e0 = jnp.exp(s0 - m0)
    sm0 = e0 / jnp.sum(e0, axis=1, keepdims=True)
    cand0 = jnp.where(in_range, sm0, -1.0)
    vals, inds = _extract_topk(cand0, piota, _K)
    vsum = vals[0]
    for t in range(1, _K):
        vsum = vsum + vals[t]
    vals = [v / vsum for v in vals]

    for lvl, kr_ref in ((1, kr1_ref), (2, kr2_ref), (3, kr3_ref)):
        qi = xq[:, lvl * _IDX_D:(lvl + 1) * _IDX_D]
        s = _dot_nt(qi, kr_ref[...]) * 0.125
        s_r = [s[:, r * _PP:(r + 1) * _PP] for r in range(_RATIO)]
        gmax = s_r[0]
        for r in range(1, _RATIO):
            gmax = jnp.maximum(gmax, s_r[r])
        e_r = [jnp.exp(sr - gmax) for sr in s_r]
        gsum = e_r[0]
        for r in range(1, _RATIO):
            gsum = gsum + e_r[r]
        w = jnp.zeros((_BT, _PP), jnp.float32)
        rank = jnp.zeros((_BT, _PP), jnp.int32)
        for t in range(_K):
            oh = piota == inds[t]
            w = w + jnp.where(oh, vals[t], 0.0)
            rank = rank + jnp.where(oh, t, 0)
        sel = w > 0.0
        cand = jnp.concatenate(
            [jnp.where(sel, w * (e_r[r] / gsum), -1.0) for r in range(_RATIO)],
            axis=1)
        codes = jnp.concatenate(
            [rank * _RATIO + r for r in range(_RATIO)], axis=1)
        vals, inds = _extract_topk(cand, codes, _K)
        if lvl < _DEPTH - 1:
            vsum = vals[0]
            for t in range(1, _K):
                vsum = vsum + vals[t]
            vals = [v / vsum for v in vals]

    oh_list = [piota == inds[t] for t in range(_K)]
    att_parts = []
    for r in range(_R):
        qr = xq[:, _DEPTH * _IDX_D + r * _MEM_D:
                _DEPTH * _IDX_D + (r + 1) * _MEM_D]
        vr = v_ref[r]
        sr = _dot_nt(qr, vr) * 0.125
        cols = [jnp.sum(jnp.where(oh, sr, 0.0), axis=1, keepdims=True)
                for oh in oh_list]
        sk = jnp.concatenate(cols, axis=1)
        mk = jnp.max(sk, axis=1, keepdims=True)
        ek = jnp.exp(sk - mk)
        wk = ek / jnp.sum(ek, axis=1, keepdims=True)
        pmat = jnp.zeros((_BT, _PP), jnp.float32)
        for t in range(_K):
            pmat = pmat + jnp.where(oh_list[t], wk[:, t:t + 1], 0.0)
        att_parts.append(_dot(pmat, vr))
    attd = jnp.concatenate(att_parts, axis=1)

    h = _dot(attd, w1_ref[...]) + b1_ref[...]
    h = 0.5 * h * (1.0 + jax.lax.erf(h * 0.7071067811865476))
    o = _dot(h, w2_ref[...]) + b2_ref[...]
    mu = jnp.mean(o, axis=1, keepdims=True)
    ctr = o - mu
    var = jnp.mean(ctr * ctr, axis=1, keepdims=True)
    out_ref[...] = g_ref[...] * ctr * jax.lax.rsqrt(var + 1e-5) + be_ref[...]


def kernel(elem_hiddens, Wq_idx, bq_idx, key0, key1, key2, key3, mem_keys,
           mem_vals, Wq_mha, bq_mha, W1, b1, W2, b2, ln_g, ln_b):
    x = elem_hiddens.reshape(_T, _HID)
    wq = jnp.concatenate([Wq_idx, Wq_mha], axis=1)
    bq = jnp.concatenate([bq_idx, bq_mha])[None, :]
    k0 = jnp.pad(key0, ((0, _PP - _P), (0, 0)))
    krs = []
    for kt in (key1, key2, key3):
        t = jnp.pad(kt[:_P].transpose(1, 0, 2), ((0, 0), (0, _PP - _P), (0, 0)))
        krs.append(t.reshape(_RATIO * _PP, _IDX_D))
    v = jnp.pad(mem_vals[:, :_P], ((0, 0), (0, _PP - _P), (0, 0)))

    full2 = lambda a: pl.BlockSpec(a.shape, lambda i: (0,) * a.ndim)
    out = pl.pallas_call(
        _fused_kernel,
        grid=(_GRID,),
        in_specs=[
            pl.BlockSpec((_BT, _HID), lambda i: (i, 0)),
            full2(wq), full2(bq), full2(k0),
            full2(krs[0]), full2(krs[1]), full2(krs[2]),
            full2(v), full2(W1), full2(b1[None, :]), full2(W2),
            full2(b2[None, :]), full2(ln_g[None, :]), full2(ln_b[None, :]),
        ],
        out_specs=pl.BlockSpec((_BT, _HID), lambda i: (i, 0)),
        out_shape=jax.ShapeDtypeStruct((_T, _HID), jnp.float32),
    )(x, wq, bq, k0, krs[0], krs[1], krs[2], v, W1, b1[None, :], W2,
      b2[None, :], ln_g[None, :], ln_b[None, :])
    return out.reshape(_B, _N, _HID)


# speedup vs baseline: 3.8424x; 3.8424x over previous
import jax
import jax.numpy as jnp
from jax.experimental import pallas as pl
from jax.experimental.pallas import tpu as pltpu

_B, _N, _HID = 2, 2048, 768
_IDX_D = 64
_K = 10
_R = 4
_MEM_D = 64
_DEPTH = 4
_P = 100
_PP = 128
_RATIO = 10

_BT = 256
_T = _B * _N
_GRID = _T // _BT

_HI = jax.lax.Precision.HIGHEST
_DF = jax.lax.Precision.DEFAULT


def _dot(a, b, precision):
    return jax.lax.dot_general(a, b, (((1,), (0,)), ((), ())),
                               precision=precision,
                               preferred_element_type=jnp.float32)


def _dot_nt(a, b, precision):
    return jax.lax.dot_general(a, b, (((1,), (1,)), ((), ())),
                               precision=precision,
                               preferred_element_type=jnp.float32)


def _tree_seq(ts):
    acc = ts[0]
    for t in ts[1:]:
        acc = acc + t
    return acc


def _tree_halve(ts):
    ts = list(ts)
    n = 1
    while n < len(ts):
        n *= 2
    ts = ts + [None] * (n - len(ts))
    while n > 1:
        n //= 2
        nxt = []
        for i in range(n):
            a, b = ts[i], ts[i + n]
            nxt.append(a if b is None else (b if a is None else a + b))
        ts = nxt
    return ts[0]


def _score_level0(q0, k0t):
    chains = []
    for c in range(16):
        acc = q0[:, c:c + 1] * k0t[c:c + 1, :]
        for i in range(1, 4):
            dd = c + 16 * i
            acc = acc + q0[:, dd:dd + 1] * k0t[dd:dd + 1, :]
        chains.append(acc)
    return _tree_halve(chains) * 0.125


def _score_chains(qi, krt):
    chunks = []
    for c in range(8):
        t = [qi[:, dd:dd + 1] * krt[dd:dd + 1, :] for dd in range(8 * c, 8 * c + 8)]
        chunks.append(((t[0] + t[1]) + (t[2] + t[3])) + ((t[4] + t[5]) + (t[6] + t[7])))
    return _tree_seq(chunks) * 0.125


def _extract_topk(cand, codes, k):
    vals, inds = [], []
    for _ in range(k):
        vmax = jnp.max(cand, axis=1, keepdims=True)
        is_max = cand == vmax
        cstar = jnp.min(jnp.where(is_max, codes, 10000), axis=1, keepdims=True)
        vals.append(vmax)
        inds.append(cstar)
        cand = jnp.where(is_max & (codes == cstar), -1.0, cand)
    return vals, inds


def _fused_kernel(x_ref, wq_ref, bq_ref, k0t_ref, kr1_ref, kr2_ref, kr3_ref,
                  v_ref, w1_ref, b1_ref, w2_ref, b2_ref, g_ref, be_ref,
                  out_ref):
    x = x_ref[...]
    xq = _dot(x, wq_ref[...], _DF) + bq_ref[...]

    piota = jax.lax.broadcasted_iota(jnp.int32, (_BT, _PP), 1)
    in_range = piota < _P

    q0 = xq[:, 0:_IDX_D]
    s0 = _score_level0(q0, k0t_ref[...])
    s0 = jnp.where(in_range, s0, -1e30)
    m0 = jnp.max(s0, axis=1, keepdims=True)
    e0 = jnp.exp(s0 - m0)
    acc = e0
    for step in range(1, 16):
        acc = acc + pltpu.roll(e0, _PP - 8 * step, axis=1)
    acc = acc + pltpu.roll(acc, _PP - 4, axis=1)
    acc = acc + pltpu.roll(acc, _PP - 2, axis=1)
    acc = acc + pltpu.roll(acc, _PP - 1, axis=1)
    d0 = acc[:, 0:1]
    sm0 = e0 / d0
    cand0 = jnp.where(in_range, sm0, -1.0)
    vals, inds = _extract_topk(cand0, piota, _K)
    vsum = _tree_halve(vals)
    vals = [v / vsum for v in vals]

    for lvl, kr_ref in ((1, kr1_ref), (2, kr2_ref), (3, kr3_ref)):
        qi = xq[:, lvl * _IDX_D:(lvl + 1) * _IDX_D]
        krt = kr_ref[...]
        s = _score_chains(qi, krt)
        s_r = [s[:, r * _PP:(r + 1) * _PP] for r in range(_RATIO)]
        gmax = s_r[0]
        for r in range(1, _RATIO):
            gmax = jnp.maximum(gmax, s_r[r])
        e_r = [jnp.exp(sr - gmax) for sr in s_r]
        gsum = _tree_seq(e_r)
        w = jnp.zeros((_BT, _PP), jnp.float32)
        rank = jnp.zeros((_BT, _PP), jnp.int32)
        for t in range(_K):
            oh = piota == inds[t]
            w = w + jnp.where(oh, vals[t], 0.0)
            rank = rank + jnp.where(oh, t, 0)
        sel = w > 0.0
        cand = jnp.concatenate(
            [jnp.where(sel, w * (e_r[r] / gsum), -1.0) for r in range(_RATIO)],
            axis=1)
        codes = jnp.concatenate(
            [rank * _RATIO + r for r in range(_RATIO)], axis=1)
        vals, inds = _extract_topk(cand, codes, _K)
        if lvl < _DEPTH - 1:
            vsum = _tree_halve(vals)
            vals = [v / vsum for v in vals]

    oh_list = [piota == inds[t] for t in range(_K)]
    att_parts = []
    for r in range(_R):
        qr = xq[:, _DEPTH * _IDX_D + r * _MEM_D:
                _DEPTH * _IDX_D + (r + 1) * _MEM_D]
        vr = v_ref[r]
        sr = _dot_nt(qr, vr, _HI) * 0.125
        cols = [jnp.sum(jnp.where(oh, sr, 0.0), axis=1, keepdims=True)
                for oh in oh_list]
        mk = cols[0]
        for c in cols[1:]:
            mk = jnp.maximum(mk, c)
        ek = [jnp.exp(c - mk) for c in cols]
        den = _tree_halve(ek)
        pmat = jnp.zeros((_BT, _PP), jnp.float32)
        for t in range(_K):
            pmat = pmat + jnp.where(oh_list[t], ek[t] / den, 0.0)
        att_parts.append(_dot(pmat, vr, _HI))
    attd = jnp.concatenate(att_parts, axis=1)

    h = _dot(attd, w1_ref[...], _DF) + b1_ref[...]
    h = 0.5 * h * (1.0 + jax.lax.erf(h * 0.7071067811865476))
    o = _dot(h, w2_ref[...], _DF) + b2_ref[...]
    mu = jnp.mean(o, axis=1, keepdims=True)
    ctr = o - mu
    var = jnp.mean(ctr * ctr, axis=1, keepdims=True)
    out_ref[...] = g_ref[...] * ctr * jax.lax.rsqrt(var + 1e-5) + be_ref[...]


def kernel(elem_hiddens, Wq_idx, bq_idx, key0, key1, key2, key3, mem_keys,
           mem_vals, Wq_mha, bq_mha, W1, b1, W2, b2, ln_g, ln_b):
    x = elem_hiddens.reshape(_T, _HID)
    wq = jnp.concatenate([Wq_idx, Wq_mha], axis=1)
    bq = jnp.concatenate([bq_idx, bq_mha])[None, :]
    k0t = jnp.pad(key0, ((0, _PP - _P), (0, 0))).T
    krs = []
    for kt in (key1, key2, key3):
        t = jnp.pad(kt[:_P].transpose(1, 0, 2), ((0, 0), (0, _PP - _P), (0, 0)))
        krs.append(t.reshape(_RATIO * _PP, _IDX_D).T)
    v = jnp.pad(mem_vals[:, :_P], ((0, 0), (0, _PP - _P), (0, 0)))

    full = lambda a: pl.BlockSpec(a.shape, lambda i: (0,) * a.ndim)
    out = pl.pallas_call(
        _fused_kernel,
        grid=(_GRID,),
        in_specs=[
            pl.BlockSpec((_BT, _HID), lambda i: (i, 0)),
            full(wq), full(bq), full(k0t),
            full(krs[0]), full(krs[1]), full(krs[2]),
            full(v), full(W1), full(b1[None, :]), full(W2),
            full(b2[None, :]), full(ln_g[None, :]), full(ln_b[None, :]),
        ],
        out_specs=pl.BlockSpec((_BT, _HID), lambda i: (i, 0)),
        out_shape=jax.ShapeDtypeStruct((_T, _HID), jnp.float32),
    )(x, wq, bq, k0t, krs[0], krs[1], krs[2], v, W1, b1[None, :], W2,
      b2[None, :], ln_g[None, :], ln_b[None, :])
    return out.reshape(_B, _N, _HID)
